# baseline (speedup 1.0000x reference)
import functools

import jax
import jax.numpy as jnp
from jax import lax
from jax.experimental import pallas as pl
from jax.experimental.pallas import tpu as pltpu
from jax.experimental.pallas import tpu_sc as plsc

VOCAB = 100000
EMBED = 64
BATCH = 1024

_NUM_CORES = 2
_NUM_SUBCORES = 16
_NUM_WORKERS = _NUM_CORES * _NUM_SUBCORES
_B_PER_W = BATCH // _NUM_WORKERS
_LANES = 16

_VT = 5120

_NBUF = 8


def _make_sc_gather():
    mesh = plsc.VectorSubcoreMesh(core_axis_name="c", subcore_axis_name="s")

    @functools.partial(
        pl.kernel,
        mesh=mesh,
        compiler_params=pltpu.CompilerParams(
            use_tc_tiling_on_sc=True, needs_layout_passes=False
        ),
        out_type=jax.ShapeDtypeStruct((BATCH, EMBED), jnp.float32),
        scratch_types=[
            pltpu.VMEM((_B_PER_W,), jnp.int32),
            pltpu.VMEM((_NBUF, EMBED, 128), jnp.float32),
            pltpu.VMEM((_B_PER_W, EMBED), jnp.float32),
            pltpu.SemaphoreType.DMA,
        ],
    )
    def gather(table_t_hbm, idx_hbm, out_hbm, idx_v, tiles_v, rows_v, sem):
        wid = lax.axis_index("s") * _NUM_CORES + lax.axis_index("c")
        base = wid * _B_PER_W
        pltpu.sync_copy(idx_hbm.at[pl.ds(base, _B_PER_W)], idx_v)

        def scalar_idx(j):
            chunk = idx_v[pl.ds((j // _LANES) * _LANES, _LANES)]
            return chunk[j % _LANES]

        def fetch(j, slot):
            t = scalar_idx(j)
            col0 = pl.multiple_of((t // 128) * 128, 128)
            return pltpu.async_copy(
                table_t_hbm.at[:, pl.ds(col0, 128)], tiles_v.at[slot], sem
            )

        inflight = [fetch(j, j % _NBUF) for j in range(_NBUF)]
        for j in range(_B_PER_W):
            inflight[j % _NBUF].wait()
            t = scalar_idx(j)
            lane = jnp.broadcast_to(t % 128, (_LANES,))
            for c in range(EMBED // _LANES):
                rows = lax.iota(jnp.int32, _LANES) + (c * _LANES)
                vals = plsc.load_gather(tiles_v.at[j % _NBUF], [rows, lane])
                rows_v[j, pl.ds(c * _LANES, _LANES)] = vals
            nxt = j + _NBUF
            if nxt < _B_PER_W:
                inflight[nxt % _NBUF] = fetch(nxt, nxt % _NBUF)
        pltpu.sync_copy(rows_v, out_hbm.at[pl.ds(base, _B_PER_W)])

    return gather


_sc_gather = _make_sc_gather()


def _proj_body(emb_ref, wt_ref, b_ref, out_ref):
    acc = lax.dot_general(
        wt_ref[...],
        emb_ref[...],
        (((0,), (1,)), ((), ())),
        preferred_element_type=jnp.float32,
    )
    out_ref[...] = acc + jnp.transpose(b_ref[...])


def _tc_project_t(embed, WT, b2d):
    grid = (pl.cdiv(VOCAB, _VT),)
    return pl.pallas_call(
        _proj_body,
        grid=grid,
        in_specs=[
            pl.BlockSpec((BATCH, EMBED), lambda i: (0, 0)),
            pl.BlockSpec((EMBED, _VT), lambda i: (0, i)),
            pl.BlockSpec((1, _VT), lambda i: (0, i)),
        ],
        out_specs=pl.BlockSpec((_VT, BATCH), lambda i: (i, 0)),
        out_shape=jax.ShapeDtypeStruct((VOCAB, BATCH), jnp.float32),
    )(embed, WT, b2d)


@jax.jit
def kernel(target, emb_table, W, b):
    embed = _sc_gather(emb_table.T, target.astype(jnp.int32))
    out_t = _tc_project_t(embed, W.T, b.reshape(1, VOCAB))
    return out_t.T


# speedup vs baseline: 1.0026x; 1.0026x over previous
import functools

import jax
import jax.numpy as jnp
from jax import lax
from jax.experimental import pallas as pl
from jax.experimental.pallas import tpu as pltpu
from jax.experimental.pallas import tpu_sc as plsc

VOCAB = 100000
EMBED = 64
BATCH = 1024

_NUM_CORES = 2
_NUM_SUBCORES = 16
_NUM_WORKERS = _NUM_CORES * _NUM_SUBCORES
_B_PER_W = BATCH // _NUM_WORKERS
_LANES = 16

_VT = 4096

_NBUF = 12


def _make_sc_gather():
    mesh = plsc.VectorSubcoreMesh(core_axis_name="c", subcore_axis_name="s")

    @functools.partial(
        pl.kernel,
        mesh=mesh,
        compiler_params=pltpu.CompilerParams(
            use_tc_tiling_on_sc=True, needs_layout_passes=False
        ),
        out_type=jax.ShapeDtypeStruct((BATCH, EMBED), jnp.float32),
        scratch_types=[
            pltpu.VMEM((_B_PER_W,), jnp.int32),
            pltpu.VMEM((_NBUF, EMBED, 128), jnp.float32),
            pltpu.VMEM((_B_PER_W, EMBED), jnp.float32),
            pltpu.SemaphoreType.DMA,
        ],
    )
    def gather(table_t_hbm, idx_hbm, out_hbm, idx_v, tiles_v, rows_v, sem):
        wid = lax.axis_index("s") * _NUM_CORES + lax.axis_index("c")
        base = wid * _B_PER_W
        pltpu.sync_copy(idx_hbm.at[pl.ds(base, _B_PER_W)], idx_v)

        def scalar_idx(j):
            chunk = idx_v[pl.ds((j // _LANES) * _LANES, _LANES)]
            return chunk[j % _LANES]

        def fetch(j, slot):
            t = scalar_idx(j)
            col0 = pl.multiple_of((t // 128) * 128, 128)
            return pltpu.async_copy(
                table_t_hbm.at[:, pl.ds(col0, 128)], tiles_v.at[slot], sem
            )

        inflight = [fetch(j, j % _NBUF) for j in range(_NBUF)]
        for j in range(_B_PER_W):
            inflight[j % _NBUF].wait()
            t = scalar_idx(j)
            lane = jnp.broadcast_to(t % 128, (_LANES,))
            for c in range(EMBED // _LANES):
                rows = lax.iota(jnp.int32, _LANES) + (c * _LANES)
                vals = plsc.load_gather(tiles_v.at[j % _NBUF], [rows, lane])
                rows_v[j, pl.ds(c * _LANES, _LANES)] = vals
            nxt = j + _NBUF
            if nxt < _B_PER_W:
                inflight[nxt % _NBUF] = fetch(nxt, nxt % _NBUF)
        pltpu.sync_copy(rows_v, out_hbm.at[pl.ds(base, _B_PER_W)])

    return gather


_sc_gather = _make_sc_gather()


def _proj_body(emb_ref, wt_ref, b_ref, out_ref):
    acc = lax.dot_general(
        wt_ref[...],
        emb_ref[...],
        (((0,), (1,)), ((), ())),
        preferred_element_type=jnp.float32,
    )
    out_ref[...] = acc + jnp.transpose(b_ref[...])


def _tc_project_t(embed, WT, b2d):
    grid = (pl.cdiv(VOCAB, _VT),)
    return pl.pallas_call(
        _proj_body,
        grid=grid,
        in_specs=[
            pl.BlockSpec((BATCH, EMBED), lambda i: (0, 0)),
            pl.BlockSpec((EMBED, _VT), lambda i: (0, i)),
            pl.BlockSpec((1, _VT), lambda i: (0, i)),
        ],
        out_specs=pl.BlockSpec((_VT, BATCH), lambda i: (i, 0)),
        out_shape=jax.ShapeDtypeStruct((VOCAB, BATCH), jnp.float32),
    )(embed, WT, b2d)


@jax.jit
def kernel(target, emb_table, W, b):
    embed = _sc_gather(emb_table.T, target.astype(jnp.int32))
    out_t = _tc_project_t(embed, W.T, b.reshape(1, VOCAB))
    return out_t.T
